# baseline (speedup 1.0000x reference)
import functools

import jax
import jax.numpy as jnp
from jax import lax
from jax.experimental import pallas as pl
from jax.experimental.pallas import tpu as pltpu
from jax.experimental.pallas import tpu_sc as plsc

NC = 2
NS = 16
LANES = 16
NW = NC * NS

TCOLS = 4096
CHUNK = 2


def _pack_body(t_ref, out_ref):
    t = t_ref[...].T
    half = TCOLS // 2
    out_ref[...] = jnp.concatenate([t[:half], t[half:]], axis=1)


def _make_pack_table(V, D):
    grid = -(-V // TCOLS)
    return pl.pallas_call(
        _pack_body,
        grid=(grid,),
        in_specs=[pl.BlockSpec((D, TCOLS), lambda g: (0, g))],
        out_specs=pl.BlockSpec((TCOLS // 2, 2 * D), lambda g: (g, 0)),
        out_shape=jax.ShapeDtypeStruct((grid * TCOLS // 2, 2 * D),
                                       jnp.float32),
        compiler_params=pltpu.CompilerParams(
            dimension_semantics=("parallel",),
            fuse_transposed_lhs_in_matmul=True),
    )


def _make_sc_gather_sum(B, L, D):
    assert B % NW == 0 and L % 2 == 0 and D % LANES == 0
    epw = B // NW
    lh = L // 2
    nd = D // LANES
    nchunks = epw // CHUNK
    assert epw % CHUNK == 0 and nchunks % 2 == 0
    mesh = plsc.VectorSubcoreMesh(core_axis_name="c", subcore_axis_name="s")

    @functools.partial(
        pl.kernel,
        mesh=mesh,
        out_type=jax.ShapeDtypeStruct((B, D), jnp.float32),
        compiler_params=pltpu.CompilerParams(use_tc_tiling_on_sc=False),
        scratch_types=[
            pltpu.VMEM((epw, 2, lh), jnp.int32),
            pltpu.VMEM((2, CHUNK, 2, lh, D), jnp.float32),
            pltpu.VMEM((epw, D), jnp.float32),
            pltpu.SemaphoreType.DMA,
            pltpu.SemaphoreType.DMA,
        ],
    )
    def sc_gather_sum(x_hbm, table_hbm, out_hbm, idx_v, rows_v, out_v,
                      sem0, sem1):
        wid = lax.axis_index("s") * NC + lax.axis_index("c")
        base = wid * epw
        sems = (sem0, sem1)

        pltpu.sync_copy(x_hbm.at[pl.ds(base, epw)], idx_v)

        def start_chunk(c, b):
            for k in range(CHUNK):
                for h in range(2):
                    pltpu.async_copy(
                        table_hbm.at[idx_v.at[c * CHUNK + k, h]],
                        rows_v.at[b, k, h], sems[b])

        def wait_chunk(c, b):
            for k in range(CHUNK):
                for h in range(2):
                    pltpu.make_async_copy(
                        table_hbm.at[idx_v.at[c * CHUNK + k, h]],
                        rows_v.at[b, k, h], sems[b]).wait()

        def reduce_chunk(c, b):
            for k in range(CHUNK):
                def body(h):
                    def red(r, acc):
                        return tuple(
                            acc[d] + rows_v[b, k, h, r,
                                            d * LANES:(d + 1) * LANES]
                            for d in range(nd)
                        )
                    return red

                acc = tuple(jnp.zeros((LANES,), jnp.float32)
                            for _ in range(nd))
                acc = lax.fori_loop(0, lh, body(0), acc, unroll=4)
                acc = lax.fori_loop(0, lh, body(1), acc, unroll=4)
                for d in range(nd):
                    out_v[c * CHUNK + k, d * LANES:(d + 1) * LANES] = acc[d]

        start_chunk(0, 0)

        def pair(q, _):
            c0 = 2 * q
            start_chunk(c0 + 1, 1)
            wait_chunk(c0, 0)
            reduce_chunk(c0, 0)
            start_chunk(c0 + 2, 0)
            wait_chunk(c0 + 1, 1)
            reduce_chunk(c0 + 1, 1)
            return ()

        lax.fori_loop(0, nchunks // 2 - 1, pair, ())
        c0 = nchunks - 2
        start_chunk(c0 + 1, 1)
        wait_chunk(c0, 0)
        reduce_chunk(c0, 0)
        wait_chunk(c0 + 1, 1)
        reduce_chunk(c0 + 1, 1)

        pltpu.sync_copy(out_v, out_hbm.at[pl.ds(base, epw)])

    return sc_gather_sum


def _mlp_body(scale, sums_ref, w1_ref, b1_ref, w2_ref, b2_ref, out_ref):
    a = sums_ref[...] * scale
    h = jnp.dot(a, w1_ref[...], preferred_element_type=jnp.float32)
    h = jnp.maximum(h + b1_ref[...], 0.0)
    o = jnp.dot(h, w2_ref[...], preferred_element_type=jnp.float32)
    o = o + b2_ref[...]
    m = jnp.max(o, axis=1, keepdims=True)
    lse = jnp.log(jnp.sum(jnp.exp(o - m), axis=1, keepdims=True)) + m
    out_ref[...] = o - lse


@jax.jit
def kernel(x, table, W1, b1, W2, b2):
    B, L = x.shape
    V, D = table.shape
    H = W1.shape[1]
    O = W2.shape[1]

    packed = _make_pack_table(V, D)(table.T)
    packed64 = packed.reshape(-1).reshape(packed.shape[0] * 2, D)

    half = TCOLS // 2
    x3 = x.reshape(B, 2, L // 2)
    gb = x3 // TCOLS
    c = x3 % TCOLS
    xrow = gb * TCOLS + 2 * (c % half) + c // half

    sums = _make_sc_gather_sum(B, L, D)(xrow, packed64)

    mlp = pl.pallas_call(
        functools.partial(_mlp_body, 1.0 / L),
        out_shape=jax.ShapeDtypeStruct((B, O), jnp.float32),
    )
    return mlp(sums, W1, b1.reshape(1, H), W2, b2.reshape(1, O))


# speedup vs baseline: 1.1600x; 1.1600x over previous
import functools

import jax
import jax.numpy as jnp
from jax import lax
from jax.experimental import pallas as pl
from jax.experimental.pallas import tpu as pltpu
from jax.experimental.pallas import tpu_sc as plsc

NC = 2
NS = 16
LANES = 16
NW = NC * NS

TCOLS = 8192
CHUNK = 2


def _pack_body(t_ref, out_ref):
    t = t_ref[...].T
    half = TCOLS // 2
    out_ref[...] = jnp.concatenate([t[:half], t[half:]], axis=1)


def _make_pack_table(V, D):
    grid = -(-V // TCOLS)
    return pl.pallas_call(
        _pack_body,
        grid=(grid,),
        in_specs=[pl.BlockSpec((D, TCOLS), lambda g: (0, g))],
        out_specs=pl.BlockSpec((TCOLS // 2, 2 * D), lambda g: (g, 0)),
        out_shape=jax.ShapeDtypeStruct((grid * TCOLS // 2, 2 * D),
                                       jnp.float32),
        compiler_params=pltpu.CompilerParams(
            dimension_semantics=("parallel",),
            fuse_transposed_lhs_in_matmul=True),
    )


def _make_sc_gather_sum(B, L, D):
    assert B % NW == 0 and L % 2 == 0 and D % LANES == 0
    epw = B // NW
    lh = L // 2
    nd = D // LANES
    nchunks = epw // CHUNK
    assert epw % CHUNK == 0 and nchunks % 2 == 0
    mesh = plsc.VectorSubcoreMesh(core_axis_name="c", subcore_axis_name="s")

    @functools.partial(
        pl.kernel,
        mesh=mesh,
        out_type=jax.ShapeDtypeStruct((B, D), jnp.float32),
        compiler_params=pltpu.CompilerParams(use_tc_tiling_on_sc=False),
        scratch_types=[
            pltpu.VMEM((epw, 2, lh), jnp.int32),
            pltpu.VMEM((2, CHUNK, 2, lh, D), jnp.float32),
            pltpu.VMEM((epw, D), jnp.float32),
            pltpu.SemaphoreType.DMA,
            pltpu.SemaphoreType.DMA,
        ],
    )
    def sc_gather_sum(x_hbm, table_hbm, out_hbm, idx_v, rows_v, out_v,
                      sem0, sem1):
        wid = lax.axis_index("s") * NC + lax.axis_index("c")
        base = wid * epw
        sems = (sem0, sem1)

        pltpu.sync_copy(x_hbm.at[pl.ds(base, epw)], idx_v)

        def start_chunk(c, b):
            for k in range(CHUNK):
                for h in range(2):
                    pltpu.async_copy(
                        table_hbm.at[idx_v.at[c * CHUNK + k, h]],
                        rows_v.at[b, k, h], sems[b])

        def wait_chunk(c, b):
            for k in range(CHUNK):
                for h in range(2):
                    pltpu.make_async_copy(
                        table_hbm.at[idx_v.at[c * CHUNK + k, h]],
                        rows_v.at[b, k, h], sems[b]).wait()

        def reduce_chunk(c, b):
            for k in range(CHUNK):
                def body(h):
                    def red(r, acc):
                        return tuple(
                            acc[d] + rows_v[b, k, h, r,
                                            d * LANES:(d + 1) * LANES]
                            for d in range(nd)
                        )
                    return red

                acc = tuple(jnp.zeros((LANES,), jnp.float32)
                            for _ in range(nd))
                acc = lax.fori_loop(0, lh, body(0), acc, unroll=4)
                acc = lax.fori_loop(0, lh, body(1), acc, unroll=4)
                for d in range(nd):
                    out_v[c * CHUNK + k, d * LANES:(d + 1) * LANES] = acc[d]

        start_chunk(0, 0)

        def pair(q, _):
            c0 = 2 * q
            start_chunk(c0 + 1, 1)
            wait_chunk(c0, 0)
            reduce_chunk(c0, 0)
            start_chunk(c0 + 2, 0)
            wait_chunk(c0 + 1, 1)
            reduce_chunk(c0 + 1, 1)
            return ()

        lax.fori_loop(0, nchunks // 2 - 1, pair, ())
        c0 = nchunks - 2
        start_chunk(c0 + 1, 1)
        wait_chunk(c0, 0)
        reduce_chunk(c0, 0)
        wait_chunk(c0 + 1, 1)
        reduce_chunk(c0 + 1, 1)

        pltpu.sync_copy(out_v, out_hbm.at[pl.ds(base, epw)])

    return sc_gather_sum


def _mlp_body(scale, sums_ref, w1_ref, b1_ref, w2_ref, b2_ref, out_ref):
    a = sums_ref[...] * scale
    h = jnp.dot(a, w1_ref[...], preferred_element_type=jnp.float32)
    h = jnp.maximum(h + b1_ref[...], 0.0)
    o = jnp.dot(h, w2_ref[...], preferred_element_type=jnp.float32)
    o = o + b2_ref[...]
    m = jnp.max(o, axis=1, keepdims=True)
    lse = jnp.log(jnp.sum(jnp.exp(o - m), axis=1, keepdims=True)) + m
    out_ref[...] = o - lse


@jax.jit
def kernel(x, table, W1, b1, W2, b2):
    B, L = x.shape
    V, D = table.shape
    H = W1.shape[1]
    O = W2.shape[1]

    packed = _make_pack_table(V, D)(table.T)
    packed64 = packed.reshape(-1).reshape(packed.shape[0] * 2, D)

    half = TCOLS // 2
    x3 = x.reshape(B, 2, L // 2)
    gb = x3 // TCOLS
    c = x3 % TCOLS
    xrow = gb * TCOLS + 2 * (c % half) + c // half

    sums = _make_sc_gather_sum(B, L, D)(xrow, packed64)

    mlp = pl.pallas_call(
        functools.partial(_mlp_body, 1.0 / L),
        out_shape=jax.ShapeDtypeStruct((B, O), jnp.float32),
    )
    return mlp(sums, W1, b1.reshape(1, H), W2, b2.reshape(1, O))


# speedup vs baseline: 1.2560x; 1.0828x over previous
import functools

import jax
import jax.numpy as jnp
from jax import lax
from jax.experimental import pallas as pl
from jax.experimental.pallas import tpu as pltpu
from jax.experimental.pallas import tpu_sc as plsc

NC = 2
NS = 16
LANES = 16
NW = NC * NS

TCOLS = 16384
CHUNK = 2


def _pack_body(t_ref, out_ref):
    t = t_ref[...].T
    half = TCOLS // 2
    out_ref[...] = jnp.concatenate([t[:half], t[half:]], axis=1)


def _make_pack_table(V, D):
    grid = -(-V // TCOLS)
    return pl.pallas_call(
        _pack_body,
        grid=(grid,),
        in_specs=[pl.BlockSpec((D, TCOLS), lambda g: (0, g))],
        out_specs=pl.BlockSpec((TCOLS // 2, 2 * D), lambda g: (g, 0)),
        out_shape=jax.ShapeDtypeStruct((grid * TCOLS // 2, 2 * D),
                                       jnp.float32),
        compiler_params=pltpu.CompilerParams(
            dimension_semantics=("parallel",),
            fuse_transposed_lhs_in_matmul=True),
    )


def _make_sc_gather_sum(B, L, D):
    assert B % NW == 0 and L % 2 == 0 and D % LANES == 0
    epw = B // NW
    lh = L // 2
    nd = D // LANES
    nchunks = epw // CHUNK
    assert epw % CHUNK == 0 and nchunks % 2 == 0
    mesh = plsc.VectorSubcoreMesh(core_axis_name="c", subcore_axis_name="s")

    @functools.partial(
        pl.kernel,
        mesh=mesh,
        out_type=jax.ShapeDtypeStruct((B, D), jnp.float32),
        compiler_params=pltpu.CompilerParams(use_tc_tiling_on_sc=False),
        scratch_types=[
            pltpu.VMEM((epw, 2, lh), jnp.int32),
            pltpu.VMEM((2, CHUNK, 2, lh, D), jnp.float32),
            pltpu.VMEM((epw, D), jnp.float32),
            pltpu.SemaphoreType.DMA,
            pltpu.SemaphoreType.DMA,
        ],
    )
    def sc_gather_sum(x_hbm, table_hbm, out_hbm, idx_v, rows_v, out_v,
                      sem0, sem1):
        wid = lax.axis_index("s") * NC + lax.axis_index("c")
        base = wid * epw
        sems = (sem0, sem1)

        pltpu.sync_copy(x_hbm.at[pl.ds(base, epw)], idx_v)

        def start_chunk(c, b):
            for k in range(CHUNK):
                for h in range(2):
                    pltpu.async_copy(
                        table_hbm.at[idx_v.at[c * CHUNK + k, h]],
                        rows_v.at[b, k, h], sems[b])

        def wait_chunk(c, b):
            for k in range(CHUNK):
                for h in range(2):
                    pltpu.make_async_copy(
                        table_hbm.at[idx_v.at[c * CHUNK + k, h]],
                        rows_v.at[b, k, h], sems[b]).wait()

        def reduce_chunk(c, b):
            for k in range(CHUNK):
                def body(h):
                    def red(r, acc):
                        return tuple(
                            acc[d] + rows_v[b, k, h, r,
                                            d * LANES:(d + 1) * LANES]
                            for d in range(nd)
                        )
                    return red

                acc = tuple(jnp.zeros((LANES,), jnp.float32)
                            for _ in range(nd))
                acc = lax.fori_loop(0, lh, body(0), acc, unroll=4)
                acc = lax.fori_loop(0, lh, body(1), acc, unroll=4)
                for d in range(nd):
                    out_v[c * CHUNK + k, d * LANES:(d + 1) * LANES] = acc[d]

        start_chunk(0, 0)

        def pair(q, _):
            c0 = 2 * q
            start_chunk(c0 + 1, 1)
            wait_chunk(c0, 0)
            reduce_chunk(c0, 0)
            start_chunk(c0 + 2, 0)
            wait_chunk(c0 + 1, 1)
            reduce_chunk(c0 + 1, 1)
            return ()

        lax.fori_loop(0, nchunks // 2 - 1, pair, ())
        c0 = nchunks - 2
        start_chunk(c0 + 1, 1)
        wait_chunk(c0, 0)
        reduce_chunk(c0, 0)
        wait_chunk(c0 + 1, 1)
        reduce_chunk(c0 + 1, 1)

        pltpu.sync_copy(out_v, out_hbm.at[pl.ds(base, epw)])

    return sc_gather_sum


def _mlp_body(scale, sums_ref, w1_ref, b1_ref, w2_ref, b2_ref, out_ref):
    a = sums_ref[...] * scale
    h = jnp.dot(a, w1_ref[...], preferred_element_type=jnp.float32)
    h = jnp.maximum(h + b1_ref[...], 0.0)
    o = jnp.dot(h, w2_ref[...], preferred_element_type=jnp.float32)
    o = o + b2_ref[...]
    m = jnp.max(o, axis=1, keepdims=True)
    lse = jnp.log(jnp.sum(jnp.exp(o - m), axis=1, keepdims=True)) + m
    out_ref[...] = o - lse


@jax.jit
def kernel(x, table, W1, b1, W2, b2):
    B, L = x.shape
    V, D = table.shape
    H = W1.shape[1]
    O = W2.shape[1]

    packed = _make_pack_table(V, D)(table.T)
    packed64 = packed.reshape(-1).reshape(packed.shape[0] * 2, D)

    half = TCOLS // 2
    x3 = x.reshape(B, 2, L // 2)
    gb = x3 // TCOLS
    c = x3 % TCOLS
    xrow = gb * TCOLS + 2 * (c % half) + c // half

    sums = _make_sc_gather_sum(B, L, D)(xrow, packed64)

    mlp = pl.pallas_call(
        functools.partial(_mlp_body, 1.0 / L),
        out_shape=jax.ShapeDtypeStruct((B, O), jnp.float32),
    )
    return mlp(sums, W1, b1.reshape(1, H), W2, b2.reshape(1, O))


# speedup vs baseline: 1.3066x; 1.0403x over previous
import functools

import jax
import jax.numpy as jnp
from jax import lax
from jax.experimental import pallas as pl
from jax.experimental.pallas import tpu as pltpu
from jax.experimental.pallas import tpu_sc as plsc

NC = 2
NS = 16
LANES = 16
NW = NC * NS

TCOLS = 32768
CHUNK = 2


def _pack_body(t_ref, out_ref):
    t = t_ref[...].T
    half = TCOLS // 2
    out_ref[...] = jnp.concatenate([t[:half], t[half:]], axis=1)


def _make_pack_table(V, D):
    grid = -(-V // TCOLS)
    return pl.pallas_call(
        _pack_body,
        grid=(grid,),
        in_specs=[pl.BlockSpec((D, TCOLS), lambda g: (0, g))],
        out_specs=pl.BlockSpec((TCOLS // 2, 2 * D), lambda g: (g, 0)),
        out_shape=jax.ShapeDtypeStruct((grid * TCOLS // 2, 2 * D),
                                       jnp.float32),
        compiler_params=pltpu.CompilerParams(
            dimension_semantics=("parallel",),
            fuse_transposed_lhs_in_matmul=True),
    )


def _make_sc_gather_sum(B, L, D):
    assert B % NW == 0 and L % 2 == 0 and D % LANES == 0
    epw = B // NW
    lh = L // 2
    nd = D // LANES
    nchunks = epw // CHUNK
    assert epw % CHUNK == 0 and nchunks % 2 == 0
    mesh = plsc.VectorSubcoreMesh(core_axis_name="c", subcore_axis_name="s")

    @functools.partial(
        pl.kernel,
        mesh=mesh,
        out_type=jax.ShapeDtypeStruct((B, D), jnp.float32),
        compiler_params=pltpu.CompilerParams(use_tc_tiling_on_sc=False),
        scratch_types=[
            pltpu.VMEM((epw, 2, lh), jnp.int32),
            pltpu.VMEM((2, CHUNK, 2, lh, D), jnp.float32),
            pltpu.VMEM((epw, D), jnp.float32),
            pltpu.SemaphoreType.DMA,
            pltpu.SemaphoreType.DMA,
        ],
    )
    def sc_gather_sum(x_hbm, table_hbm, out_hbm, idx_v, rows_v, out_v,
                      sem0, sem1):
        wid = lax.axis_index("s") * NC + lax.axis_index("c")
        base = wid * epw
        sems = (sem0, sem1)

        pltpu.sync_copy(x_hbm.at[pl.ds(base, epw)], idx_v)

        def start_chunk(c, b):
            for k in range(CHUNK):
                for h in range(2):
                    pltpu.async_copy(
                        table_hbm.at[idx_v.at[c * CHUNK + k, h]],
                        rows_v.at[b, k, h], sems[b])

        def wait_chunk(c, b):
            for k in range(CHUNK):
                for h in range(2):
                    pltpu.make_async_copy(
                        table_hbm.at[idx_v.at[c * CHUNK + k, h]],
                        rows_v.at[b, k, h], sems[b]).wait()

        def reduce_chunk(c, b):
            for k in range(CHUNK):
                def body(h):
                    def red(r, acc):
                        return tuple(
                            acc[d] + rows_v[b, k, h, r,
                                            d * LANES:(d + 1) * LANES]
                            for d in range(nd)
                        )
                    return red

                acc = tuple(jnp.zeros((LANES,), jnp.float32)
                            for _ in range(nd))
                acc = lax.fori_loop(0, lh, body(0), acc, unroll=4)
                acc = lax.fori_loop(0, lh, body(1), acc, unroll=4)
                for d in range(nd):
                    out_v[c * CHUNK + k, d * LANES:(d + 1) * LANES] = acc[d]

        start_chunk(0, 0)

        def pair(q, _):
            c0 = 2 * q
            start_chunk(c0 + 1, 1)
            wait_chunk(c0, 0)
            reduce_chunk(c0, 0)
            start_chunk(c0 + 2, 0)
            wait_chunk(c0 + 1, 1)
            reduce_chunk(c0 + 1, 1)
            return ()

        lax.fori_loop(0, nchunks // 2 - 1, pair, ())
        c0 = nchunks - 2
        start_chunk(c0 + 1, 1)
        wait_chunk(c0, 0)
        reduce_chunk(c0, 0)
        wait_chunk(c0 + 1, 1)
        reduce_chunk(c0 + 1, 1)

        pltpu.sync_copy(out_v, out_hbm.at[pl.ds(base, epw)])

    return sc_gather_sum


def _mlp_body(scale, sums_ref, w1_ref, b1_ref, w2_ref, b2_ref, out_ref):
    a = sums_ref[...] * scale
    h = jnp.dot(a, w1_ref[...], preferred_element_type=jnp.float32)
    h = jnp.maximum(h + b1_ref[...], 0.0)
    o = jnp.dot(h, w2_ref[...], preferred_element_type=jnp.float32)
    o = o + b2_ref[...]
    m = jnp.max(o, axis=1, keepdims=True)
    lse = jnp.log(jnp.sum(jnp.exp(o - m), axis=1, keepdims=True)) + m
    out_ref[...] = o - lse


@jax.jit
def kernel(x, table, W1, b1, W2, b2):
    B, L = x.shape
    V, D = table.shape
    H = W1.shape[1]
    O = W2.shape[1]

    packed = _make_pack_table(V, D)(table.T)
    packed64 = packed.reshape(-1).reshape(packed.shape[0] * 2, D)

    half = TCOLS // 2
    x3 = x.reshape(B, 2, L // 2)
    gb = x3 // TCOLS
    c = x3 % TCOLS
    xrow = gb * TCOLS + 2 * (c % half) + c // half

    sums = _make_sc_gather_sum(B, L, D)(xrow, packed64)

    mlp = pl.pallas_call(
        functools.partial(_mlp_body, 1.0 / L),
        out_shape=jax.ShapeDtypeStruct((B, O), jnp.float32),
    )
    return mlp(sums, W1, b1.reshape(1, H), W2, b2.reshape(1, O))


# speedup vs baseline: 1.3088x; 1.0017x over previous
import functools

import jax
import jax.numpy as jnp
from jax import lax
from jax.experimental import pallas as pl
from jax.experimental.pallas import tpu as pltpu
from jax.experimental.pallas import tpu_sc as plsc

NC = 2
NS = 16
LANES = 16
NW = NC * NS

TCOLS = 32768
CHUNK = 2


def _pack_body(t_ref, out_ref):
    t = t_ref[...].T
    half = TCOLS // 2
    out_ref[...] = jnp.concatenate([t[:half], t[half:]], axis=1)


def _make_pack_table(V, D):
    grid = -(-V // TCOLS)
    return pl.pallas_call(
        _pack_body,
        grid=(grid,),
        in_specs=[pl.BlockSpec((D, TCOLS), lambda g: (0, g))],
        out_specs=pl.BlockSpec((TCOLS // 2, 2 * D), lambda g: (g, 0)),
        out_shape=jax.ShapeDtypeStruct((grid * TCOLS // 2, 2 * D),
                                       jnp.float32),
        compiler_params=pltpu.CompilerParams(
            dimension_semantics=("parallel",)),
    )


def _make_sc_gather_sum(B, L, D):
    assert B % NW == 0 and L % 2 == 0 and D % LANES == 0
    epw = B // NW
    lh = L // 2
    nd = D // LANES
    nchunks = epw // CHUNK
    assert epw % CHUNK == 0 and nchunks % 2 == 0
    mesh = plsc.VectorSubcoreMesh(core_axis_name="c", subcore_axis_name="s")

    @functools.partial(
        pl.kernel,
        mesh=mesh,
        out_type=jax.ShapeDtypeStruct((B, D), jnp.float32),
        compiler_params=pltpu.CompilerParams(use_tc_tiling_on_sc=False),
        scratch_types=[
            pltpu.VMEM((epw, 2, lh), jnp.int32),
            pltpu.VMEM((2, CHUNK, 2, lh, D), jnp.float32),
            pltpu.VMEM((epw, D), jnp.float32),
            pltpu.SemaphoreType.DMA,
            pltpu.SemaphoreType.DMA,
        ],
    )
    def sc_gather_sum(x_hbm, table_hbm, out_hbm, idx_v, rows_v, out_v,
                      sem0, sem1):
        wid = lax.axis_index("s") * NC + lax.axis_index("c")
        base = wid * epw
        sems = (sem0, sem1)

        pltpu.sync_copy(x_hbm.at[pl.ds(base, epw)], idx_v)

        def start_chunk(c, b):
            for k in range(CHUNK):
                for h in range(2):
                    pltpu.async_copy(
                        table_hbm.at[idx_v.at[c * CHUNK + k, h]],
                        rows_v.at[b, k, h], sems[b])

        def wait_chunk(c, b):
            for k in range(CHUNK):
                for h in range(2):
                    pltpu.make_async_copy(
                        table_hbm.at[idx_v.at[c * CHUNK + k, h]],
                        rows_v.at[b, k, h], sems[b]).wait()

        def reduce_chunk(c, b):
            for k in range(CHUNK):
                def body(h):
                    def red(r, acc):
                        return tuple(
                            acc[d] + rows_v[b, k, h, r,
                                            d * LANES:(d + 1) * LANES]
                            for d in range(nd)
                        )
                    return red

                acc = tuple(jnp.zeros((LANES,), jnp.float32)
                            for _ in range(nd))
                acc = lax.fori_loop(0, lh, body(0), acc, unroll=4)
                acc = lax.fori_loop(0, lh, body(1), acc, unroll=4)
                for d in range(nd):
                    out_v[c * CHUNK + k, d * LANES:(d + 1) * LANES] = acc[d]

        start_chunk(0, 0)

        def pair(q, _):
            c0 = 2 * q
            start_chunk(c0 + 1, 1)
            wait_chunk(c0, 0)
            reduce_chunk(c0, 0)
            start_chunk(c0 + 2, 0)
            wait_chunk(c0 + 1, 1)
            reduce_chunk(c0 + 1, 1)
            return ()

        lax.fori_loop(0, nchunks // 2 - 1, pair, ())
        c0 = nchunks - 2
        start_chunk(c0 + 1, 1)
        wait_chunk(c0, 0)
        reduce_chunk(c0, 0)
        wait_chunk(c0 + 1, 1)
        reduce_chunk(c0 + 1, 1)

        pltpu.sync_copy(out_v, out_hbm.at[pl.ds(base, epw)])

    return sc_gather_sum


def _mlp_body(scale, sums_ref, w1_ref, b1_ref, w2_ref, b2_ref, out_ref):
    a = sums_ref[...] * scale
    h = jnp.dot(a, w1_ref[...], preferred_element_type=jnp.float32)
    h = jnp.maximum(h + b1_ref[...], 0.0)
    o = jnp.dot(h, w2_ref[...], preferred_element_type=jnp.float32)
    o = o + b2_ref[...]
    m = jnp.max(o, axis=1, keepdims=True)
    lse = jnp.log(jnp.sum(jnp.exp(o - m), axis=1, keepdims=True)) + m
    out_ref[...] = o - lse


@jax.jit
def kernel(x, table, W1, b1, W2, b2):
    B, L = x.shape
    V, D = table.shape
    H = W1.shape[1]
    O = W2.shape[1]

    packed = _make_pack_table(V, D)(table.T)
    packed64 = packed.reshape(-1).reshape(packed.shape[0] * 2, D)

    half = TCOLS // 2
    x3 = x.reshape(B, 2, L // 2)
    gb = x3 // TCOLS
    c = x3 % TCOLS
    xrow = gb * TCOLS + 2 * (c % half) + c // half

    sums = _make_sc_gather_sum(B, L, D)(xrow, packed64)

    mlp = pl.pallas_call(
        functools.partial(_mlp_body, 1.0 / L),
        out_shape=jax.ShapeDtypeStruct((B, O), jnp.float32),
    )
    return mlp(sums, W1, b1.reshape(1, H), W2, b2.reshape(1, O))


# speedup vs baseline: 1.4587x; 1.1146x over previous
import functools

import jax
import jax.numpy as jnp
from jax import lax
from jax.experimental import pallas as pl
from jax.experimental.pallas import tpu as pltpu
from jax.experimental.pallas import tpu_sc as plsc

NC = 2
NS = 16
LANES = 16
NW = NC * NS

TCOLS = 32768
CHUNK = 2


def _pack_body(t_ref, out_ref):
    t = t_ref[...].T
    half = TCOLS // 2
    out_ref[...] = jnp.concatenate([t[:half], t[half:]], axis=1)


def _make_pack_table(V, D):
    grid = -(-V // TCOLS)
    return pl.pallas_call(
        _pack_body,
        grid=(grid,),
        in_specs=[pl.BlockSpec((D, TCOLS), lambda g: (0, g))],
        out_specs=pl.BlockSpec((TCOLS // 2, 2 * D), lambda g: (g, 0)),
        out_shape=jax.ShapeDtypeStruct((grid * TCOLS // 2, 2 * D),
                                       jnp.float32),
        compiler_params=pltpu.CompilerParams(
            dimension_semantics=("parallel",)),
    )


def _make_sc_gather_sum(B, L, D):
    assert B % NW == 0 and D % LANES == 0
    epw = B // NW
    nd = D // LANES
    nchunks = epw // CHUNK
    assert epw % CHUNK == 0 and nchunks % 2 == 0
    lga = (L // 2 + 7) & ~7
    lgb = L - lga
    nidx = epw * L
    assert (L * CHUNK) % 8 == 0 and nidx % LANES == 0
    half = TCOLS // 2
    mesh = plsc.VectorSubcoreMesh(core_axis_name="c", subcore_axis_name="s")

    @functools.partial(
        pl.kernel,
        mesh=mesh,
        out_type=jax.ShapeDtypeStruct((B, D), jnp.float32),
        compiler_params=pltpu.CompilerParams(use_tc_tiling_on_sc=False),
        scratch_types=[
            pltpu.VMEM((nidx,), jnp.int32),
            pltpu.VMEM((2, CHUNK, L, D), jnp.float32),
            pltpu.VMEM((epw, D), jnp.float32),
            pltpu.SemaphoreType.DMA,
            pltpu.SemaphoreType.DMA,
        ],
    )
    def sc_gather_sum(x_hbm, table_hbm, out_hbm, idx_v, rows_v, out_v,
                      sem0, sem1):
        wid = lax.axis_index("s") * NC + lax.axis_index("c")
        base = wid * epw
        sems = (sem0, sem1)

        pltpu.sync_copy(x_hbm.at[pl.ds(base * L, nidx)], idx_v)

        shift = half.bit_length() - 1

        def xform(r, _):
            v = idx_v[pl.ds(r * LANES, LANES)]
            c = v & (TCOLS - 1)
            idx_v[pl.ds(r * LANES, LANES)] = (
                (v - c) + 2 * (c & (half - 1)) + (c >> shift))
            return ()

        lax.fori_loop(0, nidx // LANES, xform, (), unroll=8)

        def start_chunk(c, b):
            for k in range(CHUNK):
                e = c * CHUNK + k
                pltpu.async_copy(
                    table_hbm.at[idx_v.at[pl.ds(e * L, lga)]],
                    rows_v.at[b, k, pl.ds(0, lga)], sems[b])
                pltpu.async_copy(
                    table_hbm.at[idx_v.at[pl.ds(e * L + lga, lgb)]],
                    rows_v.at[b, k, pl.ds(lga, lgb)], sems[b])

        def wait_chunk(c, b):
            for k in range(CHUNK):
                e = c * CHUNK + k
                pltpu.make_async_copy(
                    table_hbm.at[idx_v.at[pl.ds(e * L, lga)]],
                    rows_v.at[b, k, pl.ds(0, lga)], sems[b]).wait()
                pltpu.make_async_copy(
                    table_hbm.at[idx_v.at[pl.ds(e * L + lga, lgb)]],
                    rows_v.at[b, k, pl.ds(lga, lgb)], sems[b]).wait()

        def reduce_chunk(c, b):
            for k in range(CHUNK):
                def red(r, acc):
                    return tuple(
                        acc[d] + rows_v[b, k, r, d * LANES:(d + 1) * LANES]
                        for d in range(nd)
                    )

                acc = tuple(jnp.zeros((LANES,), jnp.float32)
                            for _ in range(nd))
                acc = lax.fori_loop(0, L, red, acc, unroll=4)
                for d in range(nd):
                    out_v[c * CHUNK + k, d * LANES:(d + 1) * LANES] = acc[d]

        start_chunk(0, 0)

        def pair(q, _):
            c0 = 2 * q
            start_chunk(c0 + 1, 1)
            wait_chunk(c0, 0)
            reduce_chunk(c0, 0)
            start_chunk(c0 + 2, 0)
            wait_chunk(c0 + 1, 1)
            reduce_chunk(c0 + 1, 1)
            return ()

        lax.fori_loop(0, nchunks // 2 - 1, pair, ())
        c0 = nchunks - 2
        start_chunk(c0 + 1, 1)
        wait_chunk(c0, 0)
        reduce_chunk(c0, 0)
        wait_chunk(c0 + 1, 1)
        reduce_chunk(c0 + 1, 1)

        pltpu.sync_copy(out_v, out_hbm.at[pl.ds(base, epw)])

    return sc_gather_sum


def _mlp_body(scale, sums_ref, w1_ref, b1_ref, w2_ref, b2_ref, out_ref):
    a = sums_ref[...] * scale
    h = jnp.dot(a, w1_ref[...], preferred_element_type=jnp.float32)
    h = jnp.maximum(h + b1_ref[...], 0.0)
    o = jnp.dot(h, w2_ref[...], preferred_element_type=jnp.float32)
    o = o + b2_ref[...]
    m = jnp.max(o, axis=1, keepdims=True)
    lse = jnp.log(jnp.sum(jnp.exp(o - m), axis=1, keepdims=True)) + m
    out_ref[...] = o - lse


@jax.jit
def kernel(x, table, W1, b1, W2, b2):
    B, L = x.shape
    V, D = table.shape
    H = W1.shape[1]
    O = W2.shape[1]

    packed = _make_pack_table(V, D)(table.T)
    packed64 = packed.reshape(-1).reshape(packed.shape[0] * 2, D)

    sums = _make_sc_gather_sum(B, L, D)(x.reshape(-1), packed64)

    mlp = pl.pallas_call(
        functools.partial(_mlp_body, 1.0 / L),
        out_shape=jax.ShapeDtypeStruct((B, O), jnp.float32),
    )
    return mlp(sums, W1, b1.reshape(1, H), W2, b2.reshape(1, O))
